# baseline (speedup 1.0000x reference)
import functools

import jax
import jax.numpy as jnp
from jax import lax
from jax.experimental import pallas as pl
from jax.experimental.pallas import tpu as pltpu
from jax.experimental.pallas import tpu_sc as plsc

N_ITEMS = 1000000
EMBED_DIM = 64
BATCH = 16384

NC = 2
NS = 16
NW = NC * NS
B_PER_W = BATCH // NW
GRP = 16
N_GROUPS = N_ITEMS // GRP
W = 32
NWIN = B_PER_W // W
LANES = 16


W = 128
NWIN = B_PER_W // W


def _sc_gather(ids_hbm_arr, table2):
    mesh = plsc.VectorSubcoreMesh(core_axis_name="c", subcore_axis_name="s")

    @functools.partial(
        pl.kernel,
        mesh=mesh,
        out_type=jax.ShapeDtypeStruct((BATCH // 2, 128), jnp.float32),
        scratch_types=[
            pltpu.VMEM((B_PER_W,), jnp.int32),
            pltpu.VMEM((NWIN, W), jnp.int32),
            pltpu.VMEM((B_PER_W,), jnp.int32),
            pltpu.VMEM((NWIN, W, 128), jnp.float32),
            pltpu.VMEM((B_PER_W // 2, 128), jnp.float32),
            [pltpu.SemaphoreType.DMA] * NWIN,
        ],
    )
    def gather_k(ids_hbm, table_hbm, out_hbm,
                 ids_v, gid_v, po_v, bufs, stage, sems):
        wid = lax.axis_index("s") * NC + lax.axis_index("c")
        base = wid * B_PER_W
        pltpu.sync_copy(ids_hbm.at[pl.ds(base, B_PER_W)], ids_v)
        for j in range(B_PER_W // LANES):
            v = ids_v[pl.ds(j * LANES, LANES)]
            win, col = divmod(j * LANES, W)
            gid_v[win, pl.ds(col, LANES)] = lax.shift_right_logical(v, 1)
            po_v[pl.ds(j * LANES, LANES)] = lax.bitwise_and(v, 1) * EMBED_DIM

        copies = [
            pltpu.async_copy(table_hbm.at[gid_v.at[w]], bufs.at[w], sems[w])
            for w in range(NWIN)
        ]
        for w in range(NWIN):
            copies[w].wait()
            for blk in range(W // LANES):
                pov = po_v[pl.ds(w * W + blk * LANES, LANES)]
                for l in range(LANES):
                    j = w * W + blk * LANES + l
                    po = pov[l]
                    for k in range(EMBED_DIM // LANES):
                        stage[j >> 1, pl.ds((j & 1) * EMBED_DIM + k * LANES,
                                            LANES)] = (
                            bufs[w, blk * LANES + l,
                                 pl.ds(po + k * LANES, LANES)])
        out_base = pl.multiple_of(base // 2, 8)
        pltpu.sync_copy(stage, out_hbm.at[pl.ds(out_base, B_PER_W // 2)])

    return gather_k(ids_hbm_arr, table2)


def _tc_body(emb_ref, feat_ref, w1t_ref, b1_ref, w2t_ref, b2_ref,
             we_ref, wf_ref, bc_ref, out_ref):
    f = feat_ref[...]
    h = jnp.maximum(
        jnp.dot(f, w1t_ref[...], preferred_element_type=jnp.float32)
        + b1_ref[...], 0.0)
    f2 = (jnp.dot(h, w2t_ref[...], preferred_element_type=jnp.float32)
          + b2_ref[...])
    o = (jnp.dot(emb_ref[...], we_ref[...], preferred_element_type=jnp.float32)
         + jnp.dot(f2, wf_ref[...], preferred_element_type=jnp.float32)
         + bc_ref[...])
    s = jnp.sum(o * o, axis=1, keepdims=True)
    out_ref[...] = o * lax.rsqrt(jnp.maximum(s, 1e-24))


def _tc_dense(emb, feats, w1t, b1r, w2t, b2r, we, wf, bcr):
    tb = 2048
    grid = BATCH // tb
    full = lambda shape: pl.BlockSpec(shape, lambda i: (0, 0))
    return pl.pallas_call(
        _tc_body,
        grid=(grid,),
        in_specs=[
            pl.BlockSpec((tb, EMBED_DIM), lambda i: (i, 0)),
            pl.BlockSpec((tb, 4), lambda i: (i, 0)),
            full((4, 32)),
            full((1, 32)),
            full((32, EMBED_DIM)),
            full((1, EMBED_DIM)),
            full((EMBED_DIM, EMBED_DIM)),
            full((EMBED_DIM, EMBED_DIM)),
            full((1, EMBED_DIM)),
        ],
        out_specs=pl.BlockSpec((tb, EMBED_DIM), lambda i: (i, 0)),
        out_shape=jax.ShapeDtypeStruct((BATCH, EMBED_DIM), jnp.float32),
    )(emb, feats, w1t, b1r, w2t, b2r, we, wf, bcr)


def kernel(item_ids, item_features, emb_table, W1, b1, W2, b2, Wc, bc):
    emb = _sc_gather(
        item_ids.astype(jnp.int32),
        emb_table.reshape(N_ITEMS // 2, 128),
    ).reshape(BATCH, EMBED_DIM)
    return _tc_dense(
        emb,
        item_features,
        W1.T,
        b1.reshape(1, 32),
        W2.T,
        b2.reshape(1, EMBED_DIM),
        Wc[:, :EMBED_DIM].T,
        Wc[:, EMBED_DIM:].T,
        bc.reshape(1, EMBED_DIM),
    )


# speedup vs baseline: 2.0419x; 2.0419x over previous
import functools

import jax
import jax.numpy as jnp
from jax import lax
from jax.experimental import pallas as pl
from jax.experimental.pallas import tpu as pltpu
from jax.experimental.pallas import tpu_sc as plsc

N_ITEMS = 1000000
EMBED_DIM = 64
BATCH = 16384

NC = 2
NS = 16
NW = NC * NS
B_PER_W = BATCH // NW
GRP = 16
N_GROUPS = N_ITEMS // GRP
W = 32
NWIN = B_PER_W // W
LANES = 16


W = 128
NWIN = B_PER_W // W


def _sc_gather(ids_hbm_arr, table2):
    mesh = plsc.VectorSubcoreMesh(core_axis_name="c", subcore_axis_name="s")

    @functools.partial(
        pl.kernel,
        mesh=mesh,
        out_type=jax.ShapeDtypeStruct((BATCH // 2, 128), jnp.float32),
        scratch_types=[
            pltpu.VMEM((B_PER_W,), jnp.int32),
            pltpu.VMEM((NWIN, W), jnp.int32),
            pltpu.VMEM((B_PER_W,), jnp.int32),
            pltpu.VMEM((NWIN, W, 128), jnp.float32),
            pltpu.VMEM((B_PER_W // 2, 128), jnp.float32),
            [pltpu.SemaphoreType.DMA] * NWIN,
        ],
    )
    def gather_k(ids_hbm, table_hbm, out_hbm,
                 ids_v, gid_v, po_v, bufs, stage, sems):
        wid = lax.axis_index("s") * NC + lax.axis_index("c")
        base = wid * B_PER_W
        pltpu.sync_copy(ids_hbm.at[pl.ds(base, B_PER_W)], ids_v)

        def prep(j, _):
            v = ids_v[pl.ds(j * LANES, LANES)]
            q = lax.div(v, REPACK_C)
            jj = v - q * REPACK_C
            h = lax.div(jj, REPACK_H)
            jr = jj - h * REPACK_H
            win = lax.shift_right_logical(j, 3)
            col = lax.bitwise_and(j, 7) * LANES
            gid_v[win, pl.ds(col, LANES)] = q * REPACK_H + jr
            po_v[pl.ds(j * LANES, LANES)] = h * EMBED_DIM
            return 0

        lax.fori_loop(0, B_PER_W // LANES, prep, 0, unroll=False)

        copies = [
            pltpu.async_copy(table_hbm.at[gid_v.at[w]], bufs.at[w], sems[w])
            for w in range(NWIN)
        ]
        for w in range(NWIN):
            copies[w].wait()

            def ext(blk, _, w=w):
                pov = po_v[pl.ds(w * W + blk * LANES, LANES)]
                for l in range(LANES):
                    po = pov[l]
                    row = blk * (LANES // 2) + (l >> 1)
                    for k in range(EMBED_DIM // LANES):
                        stage[w * (W // 2) + row,
                              pl.ds((l & 1) * EMBED_DIM + k * LANES,
                                    LANES)] = (
                            bufs[w, blk * LANES + l,
                                 pl.ds(po + k * LANES, LANES)])
                return 0

            lax.fori_loop(0, W // LANES, ext, 0, unroll=False)
        out_base = pl.multiple_of(base // 2, 8)
        pltpu.sync_copy(stage, out_hbm.at[pl.ds(out_base, B_PER_W // 2)])

    return gather_k(ids_hbm_arr, table2)


REPACK_C = 7936
REPACK_H = REPACK_C // 2
N_STEPS = -(-N_ITEMS // REPACK_C)
T2_ROWS = N_STEPS * REPACK_H


def _repack_body(tt_ref, out_ref):
    x = tt_ref[...]
    out_ref[:, :EMBED_DIM] = x[:, :REPACK_H].T
    out_ref[:, EMBED_DIM:] = x[:, REPACK_H:].T


def _tc_repack(table_t):
    return pl.pallas_call(
        _repack_body,
        grid=(N_STEPS,),
        in_specs=[pl.BlockSpec((EMBED_DIM, REPACK_C), lambda i: (0, i))],
        out_specs=pl.BlockSpec((REPACK_H, 128), lambda i: (i, 0)),
        out_shape=jax.ShapeDtypeStruct((T2_ROWS, 128), jnp.float32),
    )(table_t)


def _tc_body(emb_ref, feat_ref, w1t_ref, b1_ref, w2t_ref, b2_ref,
             we_ref, wf_ref, bc_ref, out_ref):
    f = feat_ref[...]
    h = jnp.maximum(
        jnp.dot(f, w1t_ref[...], preferred_element_type=jnp.float32)
        + b1_ref[...], 0.0)
    f2 = (jnp.dot(h, w2t_ref[...], preferred_element_type=jnp.float32)
          + b2_ref[...])
    o = (jnp.dot(emb_ref[...], we_ref[...], preferred_element_type=jnp.float32)
         + jnp.dot(f2, wf_ref[...], preferred_element_type=jnp.float32)
         + bc_ref[...])
    s = jnp.sum(o * o, axis=1, keepdims=True)
    out_ref[...] = o * lax.rsqrt(jnp.maximum(s, 1e-24))


def _tc_dense(emb, feats, w1t, b1r, w2t, b2r, we, wf, bcr):
    tb = 2048
    grid = BATCH // tb
    full = lambda shape: pl.BlockSpec(shape, lambda i: (0, 0))
    return pl.pallas_call(
        _tc_body,
        grid=(grid,),
        in_specs=[
            pl.BlockSpec((tb, EMBED_DIM), lambda i: (i, 0)),
            pl.BlockSpec((tb, 4), lambda i: (i, 0)),
            full((4, 32)),
            full((1, 32)),
            full((32, EMBED_DIM)),
            full((1, EMBED_DIM)),
            full((EMBED_DIM, EMBED_DIM)),
            full((EMBED_DIM, EMBED_DIM)),
            full((1, EMBED_DIM)),
        ],
        out_specs=pl.BlockSpec((tb, EMBED_DIM), lambda i: (i, 0)),
        out_shape=jax.ShapeDtypeStruct((BATCH, EMBED_DIM), jnp.float32),
    )(emb, feats, w1t, b1r, w2t, b2r, we, wf, bcr)


def kernel(item_ids, item_features, emb_table, W1, b1, W2, b2, Wc, bc):
    table2 = _tc_repack(emb_table.T)
    emb = _sc_gather(item_ids.astype(jnp.int32), table2).reshape(
        BATCH, EMBED_DIM)

    return _tc_dense(
        emb,
        item_features,
        W1.T,
        b1.reshape(1, 32),
        W2.T,
        b2.reshape(1, EMBED_DIM),
        Wc[:, :EMBED_DIM].T,
        Wc[:, EMBED_DIM:].T,
        bc.reshape(1, EMBED_DIM),
    )


# speedup vs baseline: 2.0776x; 1.0174x over previous
import functools

import jax
import jax.numpy as jnp
from jax import lax
from jax.experimental import pallas as pl
from jax.experimental.pallas import tpu as pltpu
from jax.experimental.pallas import tpu_sc as plsc

N_ITEMS = 1000000
EMBED_DIM = 64
BATCH = 16384

NC = 2
NS = 16
NW = NC * NS
B_PER_W = BATCH // NW
LANES = 16
W = 128
NWIN = B_PER_W // W

REPACK_C = 7936
REPACK_H = REPACK_C // 2
N_STEPS = -(-N_ITEMS // REPACK_C)
T2_ROWS = N_STEPS * REPACK_H


def _repack_body(tt_ref, out_ref):
    x = tt_ref[...]
    out_ref[:, :EMBED_DIM] = x[:, :REPACK_H].T
    out_ref[:, EMBED_DIM:] = x[:, REPACK_H:].T


def _tc_repack(table_t):
    return pl.pallas_call(
        _repack_body,
        grid=(N_STEPS,),
        in_specs=[pl.BlockSpec((EMBED_DIM, REPACK_C), lambda i: (0, i))],
        out_specs=pl.BlockSpec((REPACK_H, 128), lambda i: (i, 0)),
        out_shape=jax.ShapeDtypeStruct((T2_ROWS, 128), jnp.float32),
    )(table_t)


def _sc_gather(gid2, po, table2):
    mesh = plsc.VectorSubcoreMesh(core_axis_name="c", subcore_axis_name="s")

    @functools.partial(
        pl.kernel,
        mesh=mesh,
        out_type=jax.ShapeDtypeStruct((BATCH, EMBED_DIM), jnp.float32),
        scratch_types=[
            pltpu.VMEM((NWIN, W), jnp.int32),
            pltpu.VMEM((B_PER_W,), jnp.int32),
            pltpu.VMEM((NWIN, W, 128), jnp.float32),
            pltpu.VMEM((W, EMBED_DIM), jnp.float32),
            [pltpu.SemaphoreType.DMA] * NWIN,
        ],
    )
    def gather_k(gid_hbm, po_hbm, table_hbm, out_hbm,
                 gid_v, po_v, bufs, stage, sems):
        wid = lax.axis_index("s") * NC + lax.axis_index("c")
        base = wid * B_PER_W
        pltpu.sync_copy(gid_hbm.at[pl.ds(wid * NWIN, NWIN)], gid_v)
        pltpu.sync_copy(po_hbm.at[pl.ds(base, B_PER_W)], po_v)

        copies = [
            pltpu.async_copy(table_hbm.at[gid_v.at[w]], bufs.at[w], sems[w])
            for w in range(NWIN)
        ]
        for w in range(NWIN):
            copies[w].wait()

            def ext(blk, _, w=w):
                pov = po_v[pl.ds(w * W + blk * LANES, LANES)]
                for l in range(LANES):
                    po_j = pov[l]
                    for k in range(EMBED_DIM // LANES):
                        stage[blk * LANES + l, pl.ds(k * LANES, LANES)] = (
                            bufs[w, blk * LANES + l,
                                 pl.ds(po_j + k * LANES, LANES)])
                return 0

            lax.fori_loop(0, W // LANES, ext, 0, unroll=False)
            out_base = pl.multiple_of(base + w * W, W)
            pltpu.sync_copy(stage, out_hbm.at[pl.ds(out_base, W)])

    return gather_k(gid2, po, table2)


def _tc_body(emb_ref, feat_ref, w1_ref, b1_ref, w2_ref, b2_ref,
             wc_ref, bc_ref, out_ref):
    dn = (((1,), (1,)), ((), ()))
    f = feat_ref[...]
    h = jnp.maximum(
        lax.dot_general(f, w1_ref[...], dn,
                        preferred_element_type=jnp.float32)
        + b1_ref[...][jnp.newaxis, :], 0.0)
    f2 = (lax.dot_general(h, w2_ref[...], dn,
                          preferred_element_type=jnp.float32)
          + b2_ref[...][jnp.newaxis, :])
    wc = wc_ref[...]
    o = (lax.dot_general(emb_ref[...], wc[:, :EMBED_DIM], dn,
                         preferred_element_type=jnp.float32)
         + lax.dot_general(f2, wc[:, EMBED_DIM:], dn,
                           preferred_element_type=jnp.float32)
         + bc_ref[...][jnp.newaxis, :])
    s = jnp.sum(o * o, axis=1, keepdims=True)
    out_ref[...] = o * lax.rsqrt(jnp.maximum(s, 1e-24))


def _tc_dense(emb, feats, W1, b1, W2, b2, Wc, bc):
    tb = 2048
    grid = BATCH // tb
    full = lambda shape: pl.BlockSpec(shape, lambda i: tuple([0] * len(shape)))
    return pl.pallas_call(
        _tc_body,
        grid=(grid,),
        in_specs=[
            pl.BlockSpec((tb, EMBED_DIM), lambda i: (i, 0)),
            pl.BlockSpec((tb, 4), lambda i: (i, 0)),
            full((32, 4)),
            full((32,)),
            full((EMBED_DIM, 32)),
            full((EMBED_DIM,)),
            full((EMBED_DIM, 2 * EMBED_DIM)),
            full((EMBED_DIM,)),
        ],
        out_specs=pl.BlockSpec((tb, EMBED_DIM), lambda i: (i, 0)),
        out_shape=jax.ShapeDtypeStruct((BATCH, EMBED_DIM), jnp.float32),
    )(emb, feats, W1, b1, W2, b2, Wc, bc)


def kernel(item_ids, item_features, emb_table, W1, b1, W2, b2, Wc, bc):
    ids = item_ids.astype(jnp.int32)
    q = ids // REPACK_C
    jj = ids - q * REPACK_C
    h = jj // REPACK_H
    jr = jj - h * REPACK_H
    gid2 = (q * REPACK_H + jr).reshape(BATCH // W, W)
    po = h * EMBED_DIM

    table2 = _tc_repack(emb_table.T)
    emb = _sc_gather(gid2, po, table2)
    return _tc_dense(emb, item_features, W1, b1, W2, b2, Wc, bc)
